# baseline (device time: 59466 ns/iter reference)
import jax
import jax.numpy as jnp
from jax import lax
from jax.experimental import pallas as pl
from jax.experimental.pallas import tpu as pltpu

N_DEV = 16
NBUF = 3
TAIL_SPLIT = 4


def kernel(x, w_mat):
    m_total, k_my = x.shape
    k_total, n = w_mat.shape
    m_blk = m_total // N_DEV
    assert k_total == N_DEV * k_my

    def body(x_ref, w_hbm, out_ref, buf_ref, send_ref, w_slots,
             send_sems, recv_sems, wdma_sems):
        me = lax.axis_index("i")

        barrier_sem = pltpu.get_barrier_semaphore()
        for d in range(1, N_DEV):
            peer = lax.rem(me + d, N_DEV)
            pl.semaphore_signal(
                barrier_sem, inc=1,
                device_id=(peer,), device_id_type=pl.DeviceIdType.MESH,
            )

        def w_dma(u):
            src = lax.rem(me + N_DEV - (u % N_DEV), N_DEV)
            return pltpu.make_async_copy(
                w_hbm.at[pl.ds(src * k_my, k_my), :],
                w_slots.at[u % NBUF],
                wdma_sems.at[u % NBUF],
            )

        for u0 in range(NBUF):
            w_dma(u0).start()

        send_ref[:, :] = x_ref[:, :].astype(jnp.bfloat16)
        buf_ref[me] = send_ref[pl.ds(me * m_blk, m_blk), :]

        w_dma(0).wait()
        out_ref[:, :] = jnp.dot(
            buf_ref[me],
            w_slots[0].astype(jnp.bfloat16),
            preferred_element_type=jnp.float32,
        )

        pl.semaphore_wait(barrier_sem, N_DEV - 1)

        sends = []
        for d in range(1, N_DEV):
            dst = lax.rem(me + d, N_DEV)
            rdma = pltpu.make_async_remote_copy(
                src_ref=send_ref.at[pl.ds(dst * m_blk, m_blk), :],
                dst_ref=buf_ref.at[me],
                send_sem=send_sems.at[d - 1],
                recv_sem=recv_sems.at[me],
                device_id=(dst,),
                device_id_type=pl.DeviceIdType.MESH,
            )
            rdma.start()
            sends.append(rdma)

        for u in range(1, N_DEV):
            if u + NBUF - 1 < N_DEV:
                w_dma(u + NBUF - 1).start()
            w_dma(u).wait()

            src = lax.rem(me + N_DEV - u, N_DEV)
            recv = pltpu.make_async_remote_copy(
                src_ref=send_ref.at[pl.ds(0, m_blk), :],
                dst_ref=buf_ref.at[src],
                send_sem=send_sems.at[0],
                recv_sem=recv_sems.at[src],
                device_id=(me,),
                device_id_type=pl.DeviceIdType.MESH,
            )
            recv.wait_recv()

            if u < N_DEV - 1:
                contrib = jnp.dot(
                    buf_ref[src],
                    w_slots[u % NBUF].astype(jnp.bfloat16),
                    preferred_element_type=jnp.float32,
                )
                out_ref[:, :] = out_ref[:, :] + contrib
            else:
                nc = n // TAIL_SPLIT
                c_gelu = 0.7978845608028654
                for c in range(TAIL_SPLIT):
                    lo, hi = c * nc, (c + 1) * nc
                    contrib = jnp.dot(
                        buf_ref[src],
                        w_slots[u % NBUF, :, lo:hi].astype(jnp.bfloat16),
                        preferred_element_type=jnp.float32,
                    )
                    y = out_ref[:, lo:hi] + contrib
                    out_ref[:, lo:hi] = 0.5 * y * (
                        1.0 + jnp.tanh(c_gelu * (y + 0.044715 * y * y * y))
                    )

        for rdma in sends:
            rdma.wait_send()

    return pl.pallas_call(
        body,
        out_shape=jax.ShapeDtypeStruct((m_blk, n), jnp.float32),
        in_specs=[
            pl.BlockSpec(memory_space=pltpu.VMEM),
            pl.BlockSpec(memory_space=pltpu.MemorySpace.HBM),
        ],
        out_specs=pl.BlockSpec(memory_space=pltpu.VMEM),
        scratch_shapes=[
            pltpu.VMEM((N_DEV, m_blk, k_my), jnp.bfloat16),
            pltpu.VMEM((m_total, k_my), jnp.bfloat16),
            pltpu.VMEM((NBUF, k_my, n), jnp.float32),
            pltpu.SemaphoreType.DMA((N_DEV - 1,)),
            pltpu.SemaphoreType.DMA((N_DEV,)),
            pltpu.SemaphoreType.DMA((NBUF,)),
        ],
        compiler_params=pltpu.CompilerParams(collective_id=0),
    )(x, w_mat)


# device time: 54857 ns/iter; 1.0840x vs baseline; 1.0840x over previous
import jax
import jax.numpy as jnp
from jax import lax
from jax.experimental import pallas as pl
from jax.experimental.pallas import tpu as pltpu

N_DEV = 16
NBUF = 3
TAIL_SPLIT = 4


def kernel(x, w_mat):
    m_total, k_my = x.shape
    k_total, n = w_mat.shape
    m_blk = m_total // N_DEV
    assert k_total == N_DEV * k_my

    def body(x_ref, w_hbm, out_ref, buf_ref, send_ref, w_slots,
             send_sems, recv_sems, wdma_sems):
        me = lax.axis_index("i")

        barrier_sem = pltpu.get_barrier_semaphore()
        for d in range(1, N_DEV):
            peer = lax.rem(me + d, N_DEV)
            pl.semaphore_signal(
                barrier_sem, inc=1,
                device_id=(peer,), device_id_type=pl.DeviceIdType.MESH,
            )

        def w_dma(u):
            src = lax.rem(me + N_DEV - (u % N_DEV), N_DEV)
            return pltpu.make_async_copy(
                w_hbm.at[pl.ds(src * k_my, k_my), :],
                w_slots.at[u % NBUF],
                wdma_sems.at[u % NBUF],
            )

        for u0 in range(NBUF):
            w_dma(u0).start()

        send_ref[:, :] = x_ref[:, :].astype(jnp.bfloat16)
        buf_ref[me] = send_ref[pl.ds(me * m_blk, m_blk), :]

        pl.semaphore_wait(barrier_sem, N_DEV - 1)

        sends = []
        for d in range(1, N_DEV):
            dst = lax.rem(me + d, N_DEV)
            rdma = pltpu.make_async_remote_copy(
                src_ref=send_ref.at[pl.ds(dst * m_blk, m_blk), :],
                dst_ref=buf_ref.at[me],
                send_sem=send_sems.at[d - 1],
                recv_sem=recv_sems.at[me],
                device_id=(dst,),
                device_id_type=pl.DeviceIdType.MESH,
            )
            rdma.start()
            sends.append(rdma)

        for u in range(N_DEV):
            if u + NBUF - 1 < N_DEV and u > 0:
                w_dma(u + NBUF - 1).start()
            w_dma(u).wait()

            src = lax.rem(me + N_DEV - u, N_DEV)
            if u > 0:
                recv = pltpu.make_async_remote_copy(
                    src_ref=send_ref.at[pl.ds(0, m_blk), :],
                    dst_ref=buf_ref.at[src],
                    send_sem=send_sems.at[0],
                    recv_sem=recv_sems.at[src],
                    device_id=(me,),
                    device_id_type=pl.DeviceIdType.MESH,
                )
                recv.wait_recv()

            if u < N_DEV - 1:
                contrib = jnp.dot(
                    buf_ref[src],
                    w_slots[u % NBUF].astype(jnp.bfloat16),
                    preferred_element_type=jnp.float32,
                )
                if u == 0:
                    out_ref[:, :] = contrib
                else:
                    out_ref[:, :] = out_ref[:, :] + contrib
            else:
                nc = n // TAIL_SPLIT
                c_gelu = 0.7978845608028654
                for c in range(TAIL_SPLIT):
                    lo, hi = c * nc, (c + 1) * nc
                    contrib = jnp.dot(
                        buf_ref[src],
                        w_slots[u % NBUF, :, lo:hi].astype(jnp.bfloat16),
                        preferred_element_type=jnp.float32,
                    )
                    y = out_ref[:, lo:hi] + contrib
                    out_ref[:, lo:hi] = 0.5 * y * (
                        1.0 + jnp.tanh(c_gelu * (y + 0.044715 * y * y * y))
                    )

        for rdma in sends:
            rdma.wait_send()

    return pl.pallas_call(
        body,
        out_shape=jax.ShapeDtypeStruct((m_blk, n), jnp.float32),
        in_specs=[
            pl.BlockSpec(memory_space=pltpu.VMEM),
            pl.BlockSpec(memory_space=pltpu.MemorySpace.HBM),
        ],
        out_specs=pl.BlockSpec(memory_space=pltpu.VMEM),
        scratch_shapes=[
            pltpu.VMEM((N_DEV, m_blk, k_my), jnp.bfloat16),
            pltpu.VMEM((m_total, k_my), jnp.bfloat16),
            pltpu.VMEM((NBUF, k_my, n), jnp.float32),
            pltpu.SemaphoreType.DMA((N_DEV - 1,)),
            pltpu.SemaphoreType.DMA((N_DEV,)),
            pltpu.SemaphoreType.DMA((NBUF,)),
        ],
        compiler_params=pltpu.CompilerParams(collective_id=0),
    )(x, w_mat)
